# baseline (speedup 1.0000x reference)
import functools

import jax
import jax.numpy as jnp
from jax import lax
from jax.experimental import pallas as pl
from jax.experimental.pallas import tpu as pltpu
from jax.experimental.pallas import tpu_sc as plsc

_B, _S, _C, _K = 32, 1024, 768, 128
_NCLS = 200
_CH = 128


def _rank_body(logits_ref, rank_ref, kc_scr, a_row_scr, c_col_scr):
    l = logits_ref[0]
    m = jnp.max(l, axis=-1, keepdims=True)
    s = jnp.sum(jnp.exp(l - m), axis=-1)
    key = 1.0 / s
    kint = lax.bitcast_convert_type(key, jnp.int32)
    for ci in range(_S // _CH):
        kc_scr[ci, :, 0] = kint[ci * _CH : (ci + 1) * _CH]
    kr = kint[None, :]

    def pass1(ci, carry):
        kc = kc_scr[ci]
        a = jnp.sum((kr > kc).astype(jnp.int32), axis=1)
        a_row_scr[ci, :] = a
        c_col_scr[ci, :, 0] = a * _S + ci * _CH + lax.broadcasted_iota(
            jnp.int32, (_CH,), 0
        )
        return carry

    lax.fori_loop(0, _S // _CH, pass1, 0)

    a_row = a_row_scr[...].reshape(1, _S)
    comp_r = a_row * _S + lax.broadcasted_iota(jnp.int32, (1, _S), 1)

    def pass2(ci, carry):
        cc = c_col_scr[ci]
        rank_ref[0, ci, :] = jnp.sum((comp_r < cc).astype(jnp.int32), axis=1)
        return carry

    lax.fori_loop(0, _S // _CH, pass2, 0)


def _token_ranks(logits):
    nch = _S // _CH
    rank3 = pl.pallas_call(
        _rank_body,
        grid=(_B,),
        in_specs=[pl.BlockSpec((1, _S, _NCLS), lambda b: (b, 0, 0))],
        out_specs=pl.BlockSpec((1, nch, _CH), lambda b: (b, 0, 0)),
        out_shape=jax.ShapeDtypeStruct((_B, nch, _CH), jnp.int32),
        scratch_shapes=[
            pltpu.VMEM((_S // _CH, _CH, 1), jnp.int32),
            pltpu.VMEM((_S // _CH, _CH), jnp.int32),
            pltpu.VMEM((_S // _CH, _CH, 1), jnp.int32),
        ],
        compiler_params=pltpu.CompilerParams(
            vmem_limit_bytes=100 * 1024 * 1024
        ),
    )(logits)
    return rank3.reshape(_B * _S)


def _make_sc_select_gather():
    info = plsc.get_sparse_core_info()
    nc = info.num_cores
    nw = nc * info.num_subcores
    mesh = plsc.VectorSubcoreMesh(core_axis_name="c", subcore_axis_name="s")

    @functools.partial(
        pl.kernel,
        mesh=mesh,
        compiler_params=pltpu.CompilerParams(needs_layout_passes=False),
        out_type=jax.ShapeDtypeStruct((_B * _K, _C), jnp.float32),
        scratch_types=[
            pltpu.VMEM((_S,), jnp.int32),
            pltpu.VMEM((_K,), jnp.int32),
            pltpu.VMEM((_K, _C), jnp.float32),
            pltpu.SemaphoreType.DMA,
        ],
    )
    def select_gather(table_hbm, rank_hbm, out_hbm, rank_v, idx_v, rows_v, sem):
        b = lax.axis_index("s") * nc + lax.axis_index("c")
        base = b * _S
        pltpu.sync_copy(rank_hbm.at[pl.ds(base, _S)], rank_v)
        lanes = lax.broadcasted_iota(jnp.int32, (16,), 0)
        for k in range(_S // 16):
            r = rank_v[pl.ds(k * 16, 16)]
            plsc.store_scatter(
                idx_v, [r], base + k * 16 + lanes, mask=r < _K
            )
        pltpu.async_copy(table_hbm.at[idx_v], rows_v, sem).wait()
        pltpu.sync_copy(rows_v, out_hbm.at[pl.ds(b * _K, _K)])

    return select_gather


def kernel(feat, logits):
    ranks = _token_ranks(logits)
    gathered = _make_sc_select_gather()(feat.reshape(_B * _S, _C), ranks)
    return gathered.reshape(_B, _K, _C)


# speedup vs baseline: 27.0452x; 27.0452x over previous
import functools

import jax
import jax.numpy as jnp
from jax import lax
from jax.experimental import pallas as pl
from jax.experimental.pallas import tpu as pltpu
from jax.experimental.pallas import tpu_sc as plsc

_B, _S, _C, _K = 32, 1024, 768, 128
_NCLS = 200
_CH = 128


def _select_body(logits_ref, idx_ref, key_scr, rank_scr):
    l = logits_ref[0]
    m = jnp.max(l, axis=-1, keepdims=True)
    s = jnp.sum(jnp.exp(l - m), axis=-1)
    key_scr[:] = 1.0 / s
    key = key_scr[:]
    b = pl.program_id(0)

    def rank_chunk(ci, carry):
        kc = key_scr[pl.ds(ci * _CH, _CH)]
        ii = ci * _CH + lax.broadcasted_iota(jnp.int32, (_CH, _S), 0)
        jj = lax.broadcasted_iota(jnp.int32, (_CH, _S), 1)
        beats = (key[None, :] > kc[:, None]) | (
            (key[None, :] == kc[:, None]) & (jj < ii)
        )
        rank_scr[pl.ds(ci * _CH, _CH)] = jnp.sum(beats.astype(jnp.int32), axis=-1)
        return carry

    lax.fori_loop(0, _S // _CH, rank_chunk, 0)
    rank = rank_scr[:]
    rr = lax.broadcasted_iota(jnp.int32, (_K, _S), 0)
    hit = rank[None, :] == rr
    jglob = b * _S + lax.broadcasted_iota(jnp.int32, (_K, _S), 1)
    idx_ref[0, 0, :] = jnp.sum(jnp.where(hit, jglob, 0), axis=-1)


def _select_indices(logits):
    idx3 = pl.pallas_call(
        _select_body,
        grid=(_B,),
        in_specs=[pl.BlockSpec((1, _S, _NCLS), lambda b: (b, 0, 0))],
        out_specs=pl.BlockSpec((1, 1, _K), lambda b: (b, 0, 0)),
        out_shape=jax.ShapeDtypeStruct((_B, 1, _K), jnp.int32),
        scratch_shapes=[
            pltpu.VMEM((_S,), jnp.float32),
            pltpu.VMEM((_S,), jnp.int32),
        ],
    )(logits)
    return idx3.reshape(_B * _K)


def _make_sc_gather():
    info = plsc.get_sparse_core_info()
    nw = info.num_cores * info.num_subcores
    rows_per_w = (_B * _K) // nw
    mesh = plsc.VectorSubcoreMesh(core_axis_name="c", subcore_axis_name="s")

    @functools.partial(
        pl.kernel,
        mesh=mesh,
        out_type=jax.ShapeDtypeStruct((_B * _K, _C), jnp.float32),
        scratch_types=[
            pltpu.VMEM((rows_per_w,), jnp.int32),
            pltpu.VMEM((rows_per_w, _C), jnp.float32),
            pltpu.SemaphoreType.DMA,
        ],
    )
    def gather(table_hbm, idx_hbm, out_hbm, idx_v, rows_v, sem):
        wid = lax.axis_index("s") * info.num_cores + lax.axis_index("c")
        base = wid * rows_per_w
        pltpu.sync_copy(idx_hbm.at[pl.ds(base, rows_per_w)], idx_v)
        pltpu.async_copy(table_hbm.at[idx_v], rows_v, sem).wait()
        pltpu.sync_copy(rows_v, out_hbm.at[pl.ds(base, rows_per_w)])

    return gather


def kernel(feat, logits):
    flat_idx = _select_indices(logits)
    gathered = _make_sc_gather()(feat.reshape(_B * _S, _C), flat_idx)
    return gathered.reshape(_B, _K, _C)
